# Initial kernel scaffold; baseline (speedup 1.0000x reference)
import functools

import jax
import jax.numpy as jnp
from jax import lax
from jax.experimental import pallas as pl
from jax.experimental.pallas import tpu as pltpu
from jax.experimental.pallas import tpu_sc as plsc

_N = 10000
_E = 160000
_F = 256
_O = 512
_NC = 2
_NS = 16
_HALF = _F // _NC
_EPT = _E // _NS
_CHUNK = 125
_NCHUNK = _EPT // _CHUNK
_RPT = _N // _NS
_DEGW = 16


def _deg_body(ei_hbm, ones_hbm, zeros_hbm, deg_hbm, idx_v, ones_v, sem, acc):
    c = lax.axis_index("c")
    s = lax.axis_index("s")
    pltpu.sync_copy(zeros_hbm, acc.at[pl.ds(s * _RPT, _RPT)])
    pltpu.sync_copy(ones_hbm, ones_v)
    pltpu.sync_copy(ei_hbm.at[1 - c, s], idx_v)
    plsc.subcore_barrier()

    def step(j, carry):
        pltpu.sync_copy(ones_v, acc.at[idx_v.at[j]], add=True)
        return carry

    lax.fori_loop(0, _NCHUNK, step, 0)
    plsc.subcore_barrier()
    pltpu.sync_copy(acc.at[pl.ds(s * _RPT, _RPT)],
                    deg_hbm.at[c, pl.ds(s * _RPT, _RPT)])


def _sc_degrees(ei4):
    ones = jnp.ones((_CHUNK, _DEGW), jnp.float32)
    zeros = jnp.zeros((_RPT, _DEGW), jnp.float32)
    mesh = plsc.VectorSubcoreMesh(core_axis_name="c", subcore_axis_name="s")
    f = pl.kernel(
        _deg_body,
        out_type=jax.ShapeDtypeStruct((_NC, _N, _DEGW), jnp.float32),
        mesh=mesh,
        scratch_types=[
            pltpu.VMEM((_NCHUNK, _CHUNK), jnp.int32),
            pltpu.VMEM((_CHUNK, _DEGW), jnp.float32),
            pltpu.SemaphoreType.DMA,
            pltpu.VMEM_SHARED((_N, _DEGW), jnp.float32),
        ],
    )
    return f(ei4, ones, zeros)


def _prescale_body(f_ref, dout_ref, h_ref):
    scale = jax.lax.rsqrt(dout_ref[...])
    x = f_ref[...] * scale
    h_ref[0, :, :] = x[:, :_HALF]
    h_ref[1, :, :] = x[:, _HALF:]


def _tc_prescale(features, dout):
    blk = 1000
    grid = (_N // blk,)
    return pl.pallas_call(
        _prescale_body,
        grid=grid,
        in_specs=[
            pl.BlockSpec((blk, _F), lambda i: (i, 0)),
            pl.BlockSpec((blk, 1), lambda i: (i, 0)),
        ],
        out_specs=pl.BlockSpec((_NC, blk, _HALF), lambda i: (0, i, 0)),
        out_shape=jax.ShapeDtypeStruct((_NC, _N, _HALF), jnp.float32),
    )(features, dout)


def _agg_body(ei_hbm, h_hbm, zeros_hbm, agg_hbm,
              src_idx, dst_idx, buf, gsem, acc):
    c = lax.axis_index("c")
    s = lax.axis_index("s")
    pltpu.sync_copy(zeros_hbm, acc.at[pl.ds(s * _RPT, _RPT)])
    pltpu.sync_copy(ei_hbm.at[0, s], src_idx)
    pltpu.sync_copy(ei_hbm.at[1, s], dst_idx)
    plsc.subcore_barrier()

    def step(j, carry):
        pltpu.async_copy(h_hbm.at[c].at[src_idx.at[j]], buf, gsem).wait()
        pltpu.sync_copy(buf, acc.at[dst_idx.at[j]], add=True)
        return carry

    lax.fori_loop(0, _NCHUNK, step, 0)
    plsc.subcore_barrier()
    pltpu.sync_copy(acc.at[pl.ds(s * _RPT, _RPT)],
                    agg_hbm.at[c, pl.ds(s * _RPT, _RPT)])


def _sc_aggregate(ei4, h3):
    zeros = jnp.zeros((_RPT, _HALF), jnp.float32)
    mesh = plsc.VectorSubcoreMesh(core_axis_name="c", subcore_axis_name="s")
    f = pl.kernel(
        _agg_body,
        out_type=jax.ShapeDtypeStruct((_NC, _N, _HALF), jnp.float32),
        mesh=mesh,
        scratch_types=[
            pltpu.VMEM((_NCHUNK, _CHUNK), jnp.int32),
            pltpu.VMEM((_NCHUNK, _CHUNK), jnp.int32),
            pltpu.VMEM((_CHUNK, _HALF), jnp.float32),
            pltpu.SemaphoreType.DMA,
            pltpu.VMEM_SHARED((_N, _HALF), jnp.float32),
        ],
    )
    return f(ei4, h3, zeros)


def _matmul_body(agg_ref, din_ref, wt_ref, b_ref, o_ref):
    x = jnp.concatenate([agg_ref[0], agg_ref[1]], axis=-1)
    x = x * jax.lax.rsqrt(din_ref[...])
    o_ref[...] = (jnp.dot(x, wt_ref[...],
                          preferred_element_type=jnp.float32)
                  + b_ref[...])


def _tc_matmul(agg3, din, W, b):
    blk = 1000
    grid = (_N // blk,)
    wt = W.T
    b2 = b.reshape(1, _O)
    return pl.pallas_call(
        _matmul_body,
        grid=grid,
        in_specs=[
            pl.BlockSpec((_NC, blk, _HALF), lambda i: (0, i, 0)),
            pl.BlockSpec((blk, 1), lambda i: (i, 0)),
            pl.BlockSpec((_F, _O), lambda i: (0, 0)),
            pl.BlockSpec((1, _O), lambda i: (0, 0)),
        ],
        out_specs=pl.BlockSpec((blk, _O), lambda i: (i, 0)),
        out_shape=jax.ShapeDtypeStruct((_N, _O), jnp.float32),
    )(agg3, din, wt, b2)


def kernel(features, edge_index, W, b):
    ei4 = edge_index.astype(jnp.int32).reshape(2, _NS, _NCHUNK, _CHUNK)
    deg = _sc_degrees(ei4)
    din = deg[0, :, :1]
    dout = deg[1, :, :1]
    h3 = _tc_prescale(features, dout)
    agg3 = _sc_aggregate(ei4, h3)
    return _tc_matmul(agg3, din, W, b)


# speedup vs baseline: 5.5274x; 5.5274x over previous
import functools

import jax
import jax.numpy as jnp
from jax import lax
from jax.experimental import pallas as pl
from jax.experimental.pallas import tpu as pltpu
from jax.experimental.pallas import tpu_sc as plsc

_N = 10000
_E = 160000
_F = 256
_O = 512
_NC = 2
_NS = 16
_HALF = _F // _NC
_EPT = _E // _NS
_CHUNK = 125
_NCHUNK = _EPT // _CHUNK
_RPT = _N // _NS
_DEGW = 16


def _deg_body(ei_hbm, ones_hbm, zeros_hbm, deg_hbm, idx_v, ones_v, sem, acc):
    c = lax.axis_index("c")
    s = lax.axis_index("s")
    pltpu.sync_copy(zeros_hbm, acc.at[pl.ds(s * _RPT, _RPT)])
    pltpu.sync_copy(ones_hbm, ones_v)
    pltpu.sync_copy(ei_hbm.at[1 - c, s], idx_v)
    plsc.subcore_barrier()

    def step(j, carry):
        pltpu.sync_copy(ones_v, acc.at[idx_v.at[j]], add=True)
        return carry

    lax.fori_loop(0, _NCHUNK, step, 0)
    plsc.subcore_barrier()
    pltpu.sync_copy(acc.at[pl.ds(s * _RPT, _RPT)], deg_hbm.at[c, s])


def _sc_degrees(ei4):
    ones = jnp.ones((_CHUNK, _DEGW), jnp.float32)
    zeros = jnp.zeros((_RPT, _DEGW), jnp.float32)
    mesh = plsc.VectorSubcoreMesh(core_axis_name="c", subcore_axis_name="s")
    f = pl.kernel(
        _deg_body,
        out_type=jax.ShapeDtypeStruct((_NC, _NS, _RPT, _DEGW), jnp.float32),
        mesh=mesh,
        scratch_types=[
            pltpu.VMEM((_NCHUNK, _CHUNK), jnp.int32),
            pltpu.VMEM((_CHUNK, _DEGW), jnp.float32),
            pltpu.SemaphoreType.DMA,
            pltpu.VMEM_SHARED((_N, _DEGW), jnp.float32),
        ],
    )
    return f(ei4, ones, zeros)


def _prescale_body(f_ref, dout_ref, h_ref):
    scale = jax.lax.rsqrt(dout_ref[...])
    x = f_ref[...] * scale
    h_ref[0, :, :] = x[:, :_HALF]
    h_ref[1, :, :] = x[:, _HALF:]


def _tc_prescale(features, dout):
    blk = 1000
    grid = (_N // blk,)
    return pl.pallas_call(
        _prescale_body,
        grid=grid,
        in_specs=[
            pl.BlockSpec((blk, _F), lambda i: (i, 0)),
            pl.BlockSpec((blk, 1), lambda i: (i, 0)),
        ],
        out_specs=pl.BlockSpec((_NC, blk, _HALF), lambda i: (0, i, 0)),
        out_shape=jax.ShapeDtypeStruct((_NC, _N, _HALF), jnp.float32),
    )(features, dout)


def _agg_body(ei_hbm, h_hbm, zeros_hbm, agg_hbm,
              src_idx, dst_idx, buf, gsem, acc):
    c = lax.axis_index("c")
    s = lax.axis_index("s")
    pltpu.sync_copy(zeros_hbm, acc.at[pl.ds(s * _RPT, _RPT)])
    pltpu.sync_copy(ei_hbm.at[0, s], src_idx)
    pltpu.sync_copy(ei_hbm.at[1, s], dst_idx)
    plsc.subcore_barrier()

    def step(j, carry):
        pltpu.async_copy(h_hbm.at[c].at[src_idx.at[j]], buf, gsem).wait()
        pltpu.sync_copy(buf, acc.at[dst_idx.at[j]], add=True)
        return carry

    lax.fori_loop(0, _NCHUNK, step, 0)
    plsc.subcore_barrier()
    pltpu.sync_copy(acc.at[pl.ds(s * _RPT, _RPT)], agg_hbm.at[c, s])


def _sc_aggregate(ei4, h3):
    zeros = jnp.zeros((_RPT, _HALF), jnp.float32)
    mesh = plsc.VectorSubcoreMesh(core_axis_name="c", subcore_axis_name="s")
    f = pl.kernel(
        _agg_body,
        out_type=jax.ShapeDtypeStruct((_NC, _NS, _RPT, _HALF), jnp.float32),
        mesh=mesh,
        scratch_types=[
            pltpu.VMEM((_NCHUNK, _CHUNK), jnp.int32),
            pltpu.VMEM((_NCHUNK, _CHUNK), jnp.int32),
            pltpu.VMEM((_CHUNK, _HALF), jnp.float32),
            pltpu.SemaphoreType.DMA,
            pltpu.VMEM_SHARED((_N, _HALF), jnp.float32),
        ],
    )
    return f(ei4, h3, zeros)


def _matmul_body(agg_ref, din_ref, wt_ref, b_ref, o_ref):
    x = jnp.concatenate([agg_ref[0], agg_ref[1]], axis=-1)
    x = x * jax.lax.rsqrt(din_ref[...])
    o_ref[...] = (jnp.dot(x, wt_ref[...],
                          preferred_element_type=jnp.float32)
                  + b_ref[...])


def _tc_matmul(agg3, din, W, b):
    blk = 1000
    grid = (_N // blk,)
    wt = W.T
    b2 = b.reshape(1, _O)
    return pl.pallas_call(
        _matmul_body,
        grid=grid,
        in_specs=[
            pl.BlockSpec((_NC, blk, _HALF), lambda i: (0, i, 0)),
            pl.BlockSpec((blk, 1), lambda i: (i, 0)),
            pl.BlockSpec((_F, _O), lambda i: (0, 0)),
            pl.BlockSpec((1, _O), lambda i: (0, 0)),
        ],
        out_specs=pl.BlockSpec((blk, _O), lambda i: (i, 0)),
        out_shape=jax.ShapeDtypeStruct((_N, _O), jnp.float32),
    )(agg3, din, wt, b2)


def kernel(features, edge_index, W, b):
    ei4 = edge_index.astype(jnp.int32).reshape(2, _NS, _NCHUNK, _CHUNK)
    deg = _sc_degrees(ei4).reshape(_NC, _N, _DEGW)
    din = deg[0, :, :1]
    dout = deg[1, :, :1]
    h3 = _tc_prescale(features, dout)
    agg3 = _sc_aggregate(ei4, h3).reshape(_NC, _N, _HALF)
    return _tc_matmul(agg3, din, W, b)
